# Initial kernel scaffold; baseline (speedup 1.0000x reference)
import jax
import jax.numpy as jnp
from jax.experimental import pallas as pl


def kernel(t, z, log_prop_dt):
    raise NotImplementedError("write your pallas kernel here")


# speedup vs baseline: 1.9531x; 1.9531x over previous
import functools

import jax
import jax.numpy as jnp
from jax import lax
from jax.experimental import pallas as pl
from jax.experimental.pallas import tpu as pltpu
from jax.experimental.pallas import tpu_sc as plsc

MAX_T = 1.0
K = 32
N_ELEMS = 8388608
NC, NS = 2, 16
NW = NC * NS
LANES = 16
CHUNK = 8192
PER_W = N_ELEMS // NW


def _butterfly(x, red_v, op):
    idx = lax.iota(jnp.int32, LANES)
    for sh in (8, 4, 2, 1):
        red_v[...] = x
        x = op(x, plsc.load_gather(red_v, [idx ^ sh]))
    return x


def _prefix_sum(x, red_v):
    idx = lax.iota(jnp.int32, LANES)
    for sh in (1, 2, 4, 8):
        red_v[...] = x
        w = plsc.load_gather(red_v, [jnp.maximum(idx - sh, 0)])
        x = x + jnp.where(idx >= sh, w, 0.0)
    return x


def _compute_tables(logdt_v, dt_v, tau_v, red_v):
    v0 = logdt_v[pl.ds(0, LANES)]
    v1 = logdt_v[pl.ds(LANES, LANES)]
    m = _butterfly(jnp.maximum(v0, v1), red_v, jnp.maximum)
    e0 = jnp.exp(v0 - m)
    e1 = jnp.exp(v1 - m)
    s = _butterfly(e0 + e1, red_v, jnp.add)
    inv = MAX_T / s
    d0 = e0 * inv
    d1 = e1 * inv
    dt_v[pl.ds(0, LANES)] = d0
    dt_v[pl.ds(LANES, LANES)] = d1
    c0 = _prefix_sum(d0, red_v)
    sum0 = _butterfly(d0, red_v, jnp.add)
    c1 = _prefix_sum(d1, red_v) + sum0
    tau_v[pl.ds(0, LANES)] = c0
    tau_v[pl.ds(LANES, LANES)] = c1


def _body(t_hbm, logdt_hbm, ind_hbm, dt_hbm, dtind_hbm, tauind_hbm,
          taunext_hbm, logdt_v, dt_v, tau_v, red_v, t_v, ind_v, dtind_v,
          tauind_v, taunext_v):
    wid = lax.axis_index("s") * NC + lax.axis_index("c")

    pltpu.sync_copy(logdt_hbm, logdt_v)
    _compute_tables(logdt_v, dt_v, tau_v, red_v)

    @pl.when(wid == 0)
    def _():
        pltpu.sync_copy(dt_v, dt_hbm)

    def chunk_body(g, _):
        base = wid * PER_W + g * CHUNK
        pltpu.sync_copy(t_hbm.at[pl.ds(base, CHUNK)], t_v)

        def vec_body(i, _):
            off = i * LANES
            t16 = t_v[pl.ds(off, LANES)]
            lo = jnp.zeros((LANES,), jnp.int32)
            for step in (16, 8, 4, 2, 1):
                probe = lo + (step - 1)
                tv = plsc.load_gather(tau_v, [probe])
                lo = lo + jnp.where(tv < t16, step, 0)
            ind = jnp.minimum(lo, K - 1)
            tnext = plsc.load_gather(tau_v, [ind])
            dti = plsc.load_gather(dt_v, [ind])
            ind_v[pl.ds(off, LANES)] = ind
            dtind_v[pl.ds(off, LANES)] = dti
            tauind_v[pl.ds(off, LANES)] = tnext - dti
            taunext_v[pl.ds(off, LANES)] = tnext
            return 0

        lax.fori_loop(0, CHUNK // LANES, vec_body, 0)

        pltpu.sync_copy(ind_v, ind_hbm.at[pl.ds(base, CHUNK)])
        pltpu.sync_copy(dtind_v, dtind_hbm.at[pl.ds(base, CHUNK)])
        pltpu.sync_copy(tauind_v, tauind_hbm.at[pl.ds(base, CHUNK)])
        pltpu.sync_copy(taunext_v, taunext_hbm.at[pl.ds(base, CHUNK)])
        return 0

    lax.fori_loop(0, PER_W // CHUNK, chunk_body, 0)


_grid_kernel = functools.partial(
    pl.kernel,
    out_type=(
        jax.ShapeDtypeStruct((N_ELEMS,), jnp.int32),
        jax.ShapeDtypeStruct((K,), jnp.float32),
        jax.ShapeDtypeStruct((N_ELEMS,), jnp.float32),
        jax.ShapeDtypeStruct((N_ELEMS,), jnp.float32),
        jax.ShapeDtypeStruct((N_ELEMS,), jnp.float32),
    ),
    mesh=plsc.VectorSubcoreMesh(core_axis_name="c", subcore_axis_name="s",
                                num_cores=NC, num_subcores=NS),
    compiler_params=pltpu.CompilerParams(needs_layout_passes=False),
    scratch_types=[
        pltpu.VMEM((K,), jnp.float32),
        pltpu.VMEM((K,), jnp.float32),
        pltpu.VMEM((K,), jnp.float32),
        pltpu.VMEM((LANES,), jnp.float32),
        pltpu.VMEM((CHUNK,), jnp.float32),
        pltpu.VMEM((CHUNK,), jnp.int32),
        pltpu.VMEM((CHUNK,), jnp.float32),
        pltpu.VMEM((CHUNK,), jnp.float32),
        pltpu.VMEM((CHUNK,), jnp.float32),
    ],
)(_body)


@jax.jit
def kernel(t, z, log_prop_dt):
    ind, dt, dt_ind, tau_ind, tau_next_ind = _grid_kernel(t, log_prop_dt)
    return (ind, dt, dt_ind, tau_ind, tau_next_ind, z)
